# Initial kernel scaffold; baseline (speedup 1.0000x reference)
import jax
import jax.numpy as jnp
from jax.experimental import pallas as pl


def kernel(uid, onehot_feats, uid_table, cat_tables, W1, b1, W2, b2):
    raise NotImplementedError("write your pallas kernel here")


# speedup vs baseline: 1.4230x; 1.4230x over previous
import jax
import jax.numpy as jnp
from jax.experimental import pallas as pl
from jax.experimental.pallas import tpu as pltpu

_N_FIELDS = 26
_EMB = 64
_DM = 512
_STEPS = _N_FIELDS + 1


def _mlp_body(idxs_ref, uid_row_ref, cat_row_ref, w1_ref, b1_ref, w2_ref,
              b2_ref, out_ref, acc_ref):
    i = pl.program_id(0)
    emb = jnp.where(i == 0, uid_row_ref[0], cat_row_ref[0, 0])
    partial = jnp.dot(emb, w1_ref[...], preferred_element_type=jnp.float32)

    @pl.when(i == 0)
    def _init():
        acc_ref[...] = partial

    @pl.when(i > 0)
    def _accum():
        acc_ref[...] += partial

    @pl.when(i == _STEPS - 1)
    def _finish():
        x = acc_ref[...] + b1_ref[...]
        x = jnp.where(x >= 0, x, 0.01 * x)
        out_ref[...] = (jnp.dot(x, w2_ref[...], preferred_element_type=jnp.float32)
                        + b2_ref[...])


def kernel(uid, onehot_feats, uid_table, cat_tables, W1, b1, W2, b2):
    idxs = jnp.concatenate(
        [uid.astype(jnp.int32), onehot_feats.reshape(-1).astype(jnp.int32)])

    grid_spec = pltpu.PrefetchScalarGridSpec(
        num_scalar_prefetch=1,
        grid=(_STEPS,),
        in_specs=[
            pl.BlockSpec((1, 1, _EMB), lambda i, idxs: (idxs[0], 0, 0)),
            pl.BlockSpec(
                (1, 1, 1, _EMB),
                lambda i, idxs: (jnp.maximum(i, 1) - 1, idxs[jnp.maximum(i, 1)],
                                 0, 0)),
            pl.BlockSpec((_EMB, _DM), lambda i, idxs: (i, 0)),
            pl.BlockSpec((1, _DM), lambda i, idxs: (0, 0)),
            pl.BlockSpec((_DM, _DM), lambda i, idxs: (0, 0)),
            pl.BlockSpec((1, _DM), lambda i, idxs: (0, 0)),
        ],
        out_specs=pl.BlockSpec((1, _DM), lambda i, idxs: (0, 0)),
        scratch_shapes=[pltpu.VMEM((1, _DM), jnp.float32)],
    )

    out = pl.pallas_call(
        _mlp_body,
        grid_spec=grid_spec,
        out_shape=jax.ShapeDtypeStruct((1, _DM), jnp.float32),
    )(idxs, uid_table.reshape(-1, 1, _EMB),
      cat_tables.reshape(_N_FIELDS, -1, 1, _EMB), W1, b1.reshape(1, -1), W2,
      b2.reshape(1, -1))
    return out[None]
